# Initial kernel scaffold; baseline (speedup 1.0000x reference)
import jax
import jax.numpy as jnp
from jax.experimental import pallas as pl


def kernel(x, edge_index, edge_attr, W):
    raise NotImplementedError("write your pallas kernel here")


# speedup vs baseline: 2.9614x; 2.9614x over previous
import functools

import jax
import jax.numpy as jnp
from jax import lax
from jax.experimental import pallas as pl
from jax.experimental.pallas import tpu as pltpu
from jax.experimental.pallas import tpu_sc as plsc

N = 10000
D_IN = 128
D_EDGE = 16
NC = 2
NS = 16
CHUNK = 128
ACC_ROWS = 10240
ROWS_PER_TILE = ACC_ROWS // NS


def _sc_aggregate(x, src, dst, ea_flat):
    e_pad = src.shape[0]
    per_tile = e_pad // NS
    n_chunks = per_tile // CHUNK

    mesh = plsc.VectorSubcoreMesh(core_axis_name="c", subcore_axis_name="s")

    @functools.partial(
        pl.kernel,
        out_type=(
            jax.ShapeDtypeStruct((ACC_ROWS, D_IN), jnp.float32),
            jax.ShapeDtypeStruct((ACC_ROWS, D_IN), jnp.float32),
        ),
        mesh=mesh,
        scratch_types=[
            pltpu.VMEM_SHARED((ACC_ROWS, D_IN), jnp.float32),
            pltpu.VMEM((1, CHUNK), jnp.int32),
            pltpu.VMEM((1, CHUNK), jnp.int32),
            pltpu.VMEM((CHUNK, D_IN), jnp.float32),
            pltpu.VMEM((CHUNK * D_EDGE,), jnp.float32),
            pltpu.SemaphoreType.DMA,
        ],
    )
    def agg(x_hbm, src_hbm, dst_hbm, ea_hbm, px_hbm, pe_hbm,
            acc, src_i, dst_i, rows, ecomp, sem):
        cid = lax.axis_index("c")
        sid = lax.axis_index("s")

        def zrow(r, _):
            for cc in range(D_IN // 16):
                rows[r, pl.ds(cc * 16, 16)] = jnp.zeros((16,), jnp.float32)
            return 0
        lax.fori_loop(0, CHUNK, zrow, 0)
        zbase = sid * ROWS_PER_TILE
        for k in range(ROWS_PER_TILE // CHUNK):
            pltpu.sync_copy(rows, acc.at[pl.ds(zbase + k * CHUNK, CHUNK)])
        plsc.subcore_barrier()

        tile_base = sid * per_tile

        @pl.when(cid == 0)
        def _():
            def chunk_body(j, _):
                base = tile_base + j * CHUNK
                pltpu.sync_copy(src_hbm.at[pl.ds(base, CHUNK)], src_i.at[0])
                pltpu.sync_copy(dst_hbm.at[pl.ds(base, CHUNK)], dst_i.at[0])
                pltpu.async_copy(x_hbm.at[src_i.at[0]], rows, sem).wait()
                pltpu.sync_copy(rows, acc.at[dst_i.at[0]], add=True)
                return 0
            lax.fori_loop(0, n_chunks, chunk_body, 0)

        @pl.when(cid == 1)
        def _():
            def chunk_body(j, _):
                base = tile_base + j * CHUNK
                pltpu.sync_copy(dst_hbm.at[pl.ds(base, CHUNK)], dst_i.at[0])
                pltpu.sync_copy(
                    ea_hbm.at[pl.ds(base * D_EDGE, CHUNK * D_EDGE)], ecomp)

                def expand(e, _):
                    rows[e, pl.ds(0, D_EDGE)] = ecomp[pl.ds(e * D_EDGE,
                                                            D_EDGE)]
                    return 0
                lax.fori_loop(0, CHUNK, expand, 0)
                pltpu.sync_copy(rows, acc.at[dst_i.at[0]], add=True)
                return 0
            lax.fori_loop(0, n_chunks, chunk_body, 0)

        plsc.subcore_barrier()

        @pl.when(cid == 0)
        def _():
            for k in range(ROWS_PER_TILE // CHUNK):
                pltpu.sync_copy(acc.at[pl.ds(zbase + k * CHUNK, CHUNK)], rows)
                pltpu.sync_copy(rows,
                                px_hbm.at[pl.ds(zbase + k * CHUNK, CHUNK)])

        @pl.when(cid == 1)
        def _():
            for k in range(ROWS_PER_TILE // CHUNK):
                pltpu.sync_copy(acc.at[pl.ds(zbase + k * CHUNK, CHUNK)], rows)
                pltpu.sync_copy(rows,
                                pe_hbm.at[pl.ds(zbase + k * CHUNK, CHUNK)])

    return agg(x, src, dst, ea_flat)


def _tc_combine(px, pe, wxt, wet_pad):
    blk = 1024
    grid = ACC_ROWS // blk

    def body(a, e, wx, we, o):
        o[...] = (jnp.dot(a[...], wx[...], preferred_element_type=jnp.float32)
                  + jnp.dot(e[...], we[...],
                            preferred_element_type=jnp.float32))

    return pl.pallas_call(
        body,
        grid=(grid,),
        in_specs=[
            pl.BlockSpec((blk, D_IN), lambda i: (i, 0)),
            pl.BlockSpec((blk, D_IN), lambda i: (i, 0)),
            pl.BlockSpec((D_IN, D_IN), lambda i: (0, 0)),
            pl.BlockSpec((D_IN, D_IN), lambda i: (0, 0)),
        ],
        out_specs=pl.BlockSpec((blk, D_IN), lambda i: (i, 0)),
        out_shape=jax.ShapeDtypeStruct((ACC_ROWS, D_IN), jnp.float32),
    )(px, pe, wxt, wet_pad)


@jax.jit
def kernel(x, edge_index, edge_attr, W):
    E = edge_index.shape[1]
    per_tile = -(-E // (NS * CHUNK)) * CHUNK
    e_pad = per_tile * NS
    pad = e_pad - E

    src = jnp.concatenate([edge_index[0], jnp.zeros((pad,), jnp.int32)])
    dst = jnp.concatenate([edge_index[1], jnp.full((pad,), N, jnp.int32)])
    ea_flat = jnp.concatenate(
        [edge_attr.reshape(-1), jnp.zeros((pad * D_EDGE,), jnp.float32)])

    px, pe = _sc_aggregate(x, src, dst, ea_flat)

    wxt = W[:, :D_IN].T
    wet_pad = jnp.concatenate(
        [W[:, D_IN:].T, jnp.zeros((D_IN - D_EDGE, D_IN), jnp.float32)])
    out = _tc_combine(px, pe, wxt, wet_pad)
    return out[:N]
